# baseline (speedup 1.0000x reference)
import functools

import jax
import jax.numpy as jnp
from jax import lax
from jax.experimental import pallas as pl
from jax.experimental.pallas import tpu as pltpu
from jax.experimental.pallas import tpu_sc as plsc

_L = 16


def _permute_cols_sc(x_flat, perm_i32, n_rows, n_cols):
    info = plsc.get_sparse_core_info()
    num_cores, num_subcores = info.num_cores, info.num_subcores
    n_workers = num_cores * num_subcores
    rows_per_w = n_rows // n_workers
    chunk_rows = 8
    n_chunks = rows_per_w // chunk_rows
    vecs_per_row = n_cols // _L
    chunk_elems = chunk_rows * n_cols

    mesh = plsc.VectorSubcoreMesh(core_axis_name="c", subcore_axis_name="s")

    @functools.partial(
        pl.kernel,
        out_type=jax.ShapeDtypeStruct((n_rows * n_cols,), jnp.float32),
        mesh=mesh,
        scratch_types=[
            pltpu.VMEM((n_cols,), jnp.int32),
            pltpu.VMEM((chunk_elems,), jnp.float32),
            pltpu.VMEM((chunk_elems,), jnp.float32),
        ],
        compiler_params=pltpu.CompilerParams(needs_layout_passes=False),
    )
    def k(x_hbm, perm_hbm, out_hbm, perm_v, in_v, out_v):
        wid = lax.axis_index("s") * num_cores + lax.axis_index("c")
        base = wid * rows_per_w * n_cols
        pltpu.sync_copy(perm_hbm, perm_v)

        def chunk_body(g, carry):
            off = base + g * chunk_elems
            pltpu.sync_copy(x_hbm.at[pl.ds(off, chunk_elems)], in_v)

            def col_body(v, carry2):
                cbase = v * _L
                col = perm_v[pl.ds(cbase, _L)]
                for r in range(chunk_rows):
                    val = plsc.load_gather(in_v, [col + r * n_cols])
                    out_v[pl.ds(r * n_cols + cbase, _L)] = val
                return carry2

            lax.fori_loop(0, vecs_per_row, col_body, 0, unroll=2)
            pltpu.sync_copy(out_v, out_hbm.at[pl.ds(off, chunk_elems)])
            return carry

        lax.fori_loop(0, n_chunks, chunk_body, 0)

    return k(x_flat, perm_i32)


def kernel(x, perm):
    n_rows, n_cols = x.shape
    out_flat = _permute_cols_sc(
        x.reshape(n_rows * n_cols), perm.astype(jnp.int32), n_rows, n_cols
    )
    return out_flat.reshape(n_rows, n_cols)


# speedup vs baseline: 1.2350x; 1.2350x over previous
import functools

import jax
import jax.numpy as jnp
from jax import lax
from jax.experimental import pallas as pl
from jax.experimental.pallas import tpu as pltpu
from jax.experimental.pallas import tpu_sc as plsc

_L = 16


def _permute_cols_sc(x_flat, perm_i32, n_rows, n_cols):
    info = plsc.get_sparse_core_info()
    num_cores, num_subcores = info.num_cores, info.num_subcores
    n_workers = num_cores * num_subcores
    rows_per_w = n_rows // n_workers
    chunk_rows = 8
    n_chunks = rows_per_w // chunk_rows
    vecs_per_row = n_cols // _L
    chunk_elems = chunk_rows * n_cols

    mesh = plsc.VectorSubcoreMesh(core_axis_name="c", subcore_axis_name="s")

    @functools.partial(
        pl.kernel,
        out_type=jax.ShapeDtypeStruct((n_rows * n_cols,), jnp.float32),
        mesh=mesh,
        scratch_types=[
            pltpu.VMEM((n_cols,), jnp.int32),
            pltpu.VMEM((chunk_elems,), jnp.float32),
            pltpu.VMEM((chunk_elems,), jnp.float32),
            pltpu.VMEM((chunk_elems,), jnp.float32),
            pltpu.VMEM((chunk_elems,), jnp.float32),
            pltpu.SemaphoreType.DMA,
            pltpu.SemaphoreType.DMA,
            pltpu.SemaphoreType.DMA,
            pltpu.SemaphoreType.DMA,
        ],
        compiler_params=pltpu.CompilerParams(needs_layout_passes=False),
    )
    def k(x_hbm, perm_hbm, out_hbm, perm_v, in0, in1, ot0, ot1, is0, is1, os0, os1):
        wid = lax.axis_index("s") * num_cores + lax.axis_index("c")
        base = wid * rows_per_w * n_cols
        in_bufs = (in0, in1)
        out_bufs = (ot0, ot1)
        in_sems = (is0, is1)
        out_sems = (os0, os1)
        pltpu.sync_copy(perm_hbm, perm_v)

        def start_in(g, b):
            pltpu.async_copy(
                x_hbm.at[pl.ds(base + g * chunk_elems, chunk_elems)], in_bufs[b],
                in_sems[b],
            )

        def wait_in(b):
            pltpu.make_async_copy(
                x_hbm.at[pl.ds(0, chunk_elems)], in_bufs[b], in_sems[b]
            ).wait()

        def start_out(g, b):
            pltpu.async_copy(
                out_bufs[b], out_hbm.at[pl.ds(base + g * chunk_elems, chunk_elems)],
                out_sems[b],
            )

        def wait_out(b):
            pltpu.make_async_copy(
                out_bufs[b], out_hbm.at[pl.ds(0, chunk_elems)], out_sems[b]
            ).wait()

        def compute(b):
            def col_body(v, carry2):
                cbase = v * _L
                col = perm_v[pl.ds(cbase, _L)]
                for r in range(chunk_rows):
                    val = plsc.load_gather(in_bufs[b], [col + r * n_cols])
                    out_bufs[b][pl.ds(r * n_cols + cbase, _L)] = val
                return carry2

            lax.fori_loop(0, vecs_per_row, col_body, 0, unroll=4)

        start_in(0, 0)
        start_in(1, 1)
        for b in range(2):
            wait_in(b)
            compute(b)
            start_out(b, b)
            start_in(b + 2, b)

        def chunk_body(i, carry):
            g0 = 2 + 2 * i
            for b in range(2):
                g = g0 + b
                wait_in(b)
                wait_out(b)
                compute(b)
                start_out(g, b)

                @pl.when(g + 2 < n_chunks)
                def _():
                    start_in(g + 2, b)

            return carry

        lax.fori_loop(0, (n_chunks - 2) // 2, chunk_body, 0, unroll=1)
        wait_out(0)
        wait_out(1)

    return k(x_flat, perm_i32)


def kernel(x, perm):
    n_rows, n_cols = x.shape
    out_flat = _permute_cols_sc(
        x.reshape(n_rows * n_cols), perm.astype(jnp.int32), n_rows, n_cols
    )
    return out_flat.reshape(n_rows, n_cols)


# speedup vs baseline: 2.0313x; 1.6447x over previous
import functools

import jax
import jax.numpy as jnp
from jax import lax
from jax.experimental import pallas as pl
from jax.experimental.pallas import tpu as pltpu
from jax.experimental.pallas import tpu_sc as plsc

_L = 16


def _permute_cols_sc(x_flat, perm_i32, n_rows, n_cols):
    info = plsc.get_sparse_core_info()
    num_cores, num_subcores = info.num_cores, info.num_subcores
    n_workers = num_cores * num_subcores
    rows_per_w = n_rows // n_workers
    chunk_rows = 8
    n_chunks = rows_per_w // chunk_rows
    vecs_per_row = n_cols // _L
    chunk_elems = chunk_rows * n_cols

    mesh = plsc.VectorSubcoreMesh(core_axis_name="c", subcore_axis_name="s")

    @functools.partial(
        pl.kernel,
        out_type=jax.ShapeDtypeStruct((n_rows * n_cols,), jnp.float32),
        mesh=mesh,
        scratch_types=[
            pltpu.VMEM((n_cols,), jnp.int32),
            pltpu.VMEM((chunk_elems,), jnp.float32),
            pltpu.VMEM((chunk_elems,), jnp.float32),
            pltpu.VMEM((chunk_elems,), jnp.float32),
            pltpu.VMEM((chunk_elems,), jnp.float32),
            pltpu.SemaphoreType.DMA,
            pltpu.SemaphoreType.DMA,
            pltpu.SemaphoreType.DMA,
            pltpu.SemaphoreType.DMA,
        ],
        compiler_params=pltpu.CompilerParams(needs_layout_passes=False),
    )
    def k(x_hbm, perm_hbm, out_hbm, perm_v, in0, in1, ot0, ot1, is0, is1, os0, os1):
        wid = lax.axis_index("s") * num_cores + lax.axis_index("c")
        base = wid * rows_per_w * n_cols
        in_bufs = (in0, in1)
        out_bufs = (ot0, ot1)
        in_sems = (is0, is1)
        out_sems = (os0, os1)
        pltpu.sync_copy(perm_hbm, perm_v)

        def start_in(g, b):
            pltpu.async_copy(
                x_hbm.at[pl.ds(base + g * chunk_elems, chunk_elems)], in_bufs[b],
                in_sems[b],
            )

        def wait_in(b):
            pltpu.make_async_copy(
                x_hbm.at[pl.ds(0, chunk_elems)], in_bufs[b], in_sems[b]
            ).wait()

        def start_out(g, b):
            pltpu.async_copy(
                out_bufs[b], out_hbm.at[pl.ds(base + g * chunk_elems, chunk_elems)],
                out_sems[b],
            )

        def wait_out(b):
            pltpu.make_async_copy(
                out_bufs[b], out_hbm.at[pl.ds(0, chunk_elems)], out_sems[b]
            ).wait()

        def compute(b):
            @plsc.parallel_loop(0, n_cols, step=_L, unroll=4)
            def col_body(cbase):
                col = perm_v[pl.ds(cbase, _L)]
                for r in range(chunk_rows):
                    val = plsc.load_gather(in_bufs[b], [col + r * n_cols])
                    out_bufs[b][pl.ds(r * n_cols + cbase, _L)] = val

        start_in(0, 0)
        start_in(1, 1)
        for b in range(2):
            wait_in(b)
            compute(b)
            start_out(b, b)
            start_in(b + 2, b)

        def chunk_body(i, carry):
            g0 = 2 + 2 * i
            for b in range(2):
                g = g0 + b
                wait_in(b)
                wait_out(b)
                compute(b)
                start_out(g, b)

                @pl.when(g + 2 < n_chunks)
                def _():
                    start_in(g + 2, b)

            return carry

        lax.fori_loop(0, (n_chunks - 2) // 2, chunk_body, 0, unroll=1)
        wait_out(0)
        wait_out(1)

    return k(x_flat, perm_i32)


def kernel(x, perm):
    n_rows, n_cols = x.shape
    out_flat = _permute_cols_sc(
        x.reshape(n_rows * n_cols), perm.astype(jnp.int32), n_rows, n_cols
    )
    return out_flat.reshape(n_rows, n_cols)


# speedup vs baseline: 2.0345x; 1.0016x over previous
import functools

import jax
import jax.numpy as jnp
from jax import lax
from jax.experimental import pallas as pl
from jax.experimental.pallas import tpu as pltpu
from jax.experimental.pallas import tpu_sc as plsc

_L = 16


def _permute_cols_sc(x_flat, perm_i32, n_rows, n_cols):
    info = plsc.get_sparse_core_info()
    num_cores, num_subcores = info.num_cores, info.num_subcores
    n_workers = num_cores * num_subcores
    rows_per_w = n_rows // n_workers
    chunk_rows = 8
    n_chunks = rows_per_w // chunk_rows
    vecs_per_row = n_cols // _L
    chunk_elems = chunk_rows * n_cols

    mesh = plsc.VectorSubcoreMesh(core_axis_name="c", subcore_axis_name="s")

    @functools.partial(
        pl.kernel,
        out_type=jax.ShapeDtypeStruct((n_rows * n_cols,), jnp.float32),
        mesh=mesh,
        scratch_types=[
            pltpu.VMEM((n_cols,), jnp.int32),
            pltpu.VMEM((chunk_elems,), jnp.float32),
            pltpu.VMEM((chunk_elems,), jnp.float32),
            pltpu.VMEM((chunk_elems,), jnp.float32),
            pltpu.VMEM((chunk_elems,), jnp.float32),
            pltpu.SemaphoreType.DMA,
            pltpu.SemaphoreType.DMA,
            pltpu.SemaphoreType.DMA,
            pltpu.SemaphoreType.DMA,
        ],
        compiler_params=pltpu.CompilerParams(needs_layout_passes=False),
    )
    def k(x_hbm, perm_hbm, out_hbm, perm_v, in0, in1, ot0, ot1, is0, is1, os0, os1):
        wid = lax.axis_index("s") * num_cores + lax.axis_index("c")
        base = wid * rows_per_w * n_cols
        in_bufs = (in0, in1)
        out_bufs = (ot0, ot1)
        in_sems = (is0, is1)
        out_sems = (os0, os1)
        pltpu.sync_copy(perm_hbm, perm_v)

        def start_in(g, b):
            pltpu.async_copy(
                x_hbm.at[pl.ds(base + g * chunk_elems, chunk_elems)], in_bufs[b],
                in_sems[b],
            )

        def wait_in(b):
            pltpu.make_async_copy(
                x_hbm.at[pl.ds(0, chunk_elems)], in_bufs[b], in_sems[b]
            ).wait()

        def start_out(g, b):
            pltpu.async_copy(
                out_bufs[b], out_hbm.at[pl.ds(base + g * chunk_elems, chunk_elems)],
                out_sems[b],
            )

        def wait_out(b):
            pltpu.make_async_copy(
                out_bufs[b], out_hbm.at[pl.ds(0, chunk_elems)], out_sems[b]
            ).wait()

        def compute(b):
            @plsc.parallel_loop(0, n_cols, step=_L, unroll=8)
            def col_body(cbase):
                col = perm_v[pl.ds(cbase, _L)]
                for r in range(chunk_rows):
                    val = plsc.load_gather(in_bufs[b], [col + r * n_cols])
                    out_bufs[b][pl.ds(r * n_cols + cbase, _L)] = val

        start_in(0, 0)
        start_in(1, 1)
        for b in range(2):
            wait_in(b)
            compute(b)
            start_out(b, b)
            start_in(b + 2, b)

        def chunk_body(i, carry):
            g0 = 2 + 2 * i
            for b in range(2):
                g = g0 + b
                wait_in(b)
                wait_out(b)
                compute(b)
                start_out(g, b)

                @pl.when(g + 2 < n_chunks)
                def _():
                    start_in(g + 2, b)

            return carry

        lax.fori_loop(0, (n_chunks - 2) // 2, chunk_body, 0, unroll=1)
        wait_out(0)
        wait_out(1)

    return k(x_flat, perm_i32)


def kernel(x, perm):
    n_rows, n_cols = x.shape
    out_flat = _permute_cols_sc(
        x.reshape(n_rows * n_cols), perm.astype(jnp.int32), n_rows, n_cols
    )
    return out_flat.reshape(n_rows, n_cols)
